# baseline (speedup 1.0000x reference)
import functools

import jax
import jax.numpy as jnp
from jax import lax
from jax.experimental import pallas as pl
from jax.experimental.pallas import tpu as pltpu
from jax.experimental.pallas import tpu_sc as plsc

N_NODES = 10000
N_EDGES = 320000
D_FEAT = 128
N_CLASSES = 16

NC = 2
NS = 16
CHUNK = 256
NCHUNKS = N_EDGES // CHUNK
CPT = 39
EXTRA = NCHUNKS - NC * NS * CPT
NODES_PAD = 10240
ROWS_PER_TILE = NODES_PAD // NS
G_ROWS_PER_TILE = N_NODES // NS

NBUF = 3
LOOKAHEAD = 2

_mesh = plsc.VectorSubcoreMesh(core_axis_name="c", subcore_axis_name="s")


def _deg_body(cidx_hbm, ones_hbm, zeros_hbm, out_hbm, cidx_v, ones_v, stage_v, acc, sem):
    cid = lax.axis_index("c")
    sid = lax.axis_index("s")
    wid = cid * NS + sid
    pltpu.sync_copy(zeros_hbm, stage_v)
    pltpu.sync_copy(stage_v, acc.at[pl.ds(sid * ROWS_PER_TILE, ROWS_PER_TILE)])
    pltpu.sync_copy(ones_hbm, ones_v)
    pltpu.sync_copy(cidx_hbm.at[pl.ds(wid * CPT, CPT)], cidx_v)
    plsc.subcore_barrier()

    def chunk(j, carry):
        pltpu.async_copy(ones_v, acc.at[cidx_v.at[j, :]], sem, add=True)
        return carry

    lax.fori_loop(0, CPT, chunk, 0)

    @pl.when(wid < EXTRA)
    def _():
        pltpu.sync_copy(cidx_hbm.at[pl.ds(NC * NS * CPT + wid, 1)],
                        cidx_v.at[pl.ds(0, 1)])
        pltpu.async_copy(ones_v, acc.at[cidx_v.at[0, :]], sem, add=True)

    def drain(j, carry):
        pltpu.make_async_copy(ones_v, acc.at[cidx_v.at[0, :]], sem).wait()
        return carry

    lax.fori_loop(0, CPT, drain, 0)

    @pl.when(wid < EXTRA)
    def _():
        pltpu.make_async_copy(ones_v, acc.at[cidx_v.at[0, :]], sem).wait()

    plsc.subcore_barrier()
    pltpu.sync_copy(acc.at[pl.ds(sid * ROWS_PER_TILE, ROWS_PER_TILE)], stage_v)
    pltpu.sync_copy(stage_v, out_hbm.at[cid, pl.ds(sid * ROWS_PER_TILE, ROWS_PER_TILE)])


def _hop_body(ridx_hbm, cidx_hbm, g_hbm, zeros_hbm, out_hbm,
              ridx_v, cidx_v, rows_v, stage_v, g_stage_v, acc, g_s, sem):
    cid = lax.axis_index("c")
    sid = lax.axis_index("s")
    wid = cid * NS + sid
    pltpu.sync_copy(zeros_hbm, stage_v)
    pltpu.sync_copy(stage_v, acc.at[pl.ds(sid * ROWS_PER_TILE, ROWS_PER_TILE)])
    pltpu.sync_copy(g_hbm.at[pl.ds(sid * G_ROWS_PER_TILE, G_ROWS_PER_TILE)],
                    g_stage_v)
    pltpu.sync_copy(g_stage_v,
                    g_s.at[pl.ds(sid * G_ROWS_PER_TILE, G_ROWS_PER_TILE)])
    pltpu.sync_copy(ridx_hbm.at[pl.ds(wid * CPT, CPT)], ridx_v)
    pltpu.sync_copy(cidx_hbm.at[pl.ds(wid * CPT, CPT)], cidx_v)
    gsem, ssem = sem
    plsc.subcore_barrier()

    for b in range(LOOKAHEAD):
        pltpu.async_copy(g_s.at[ridx_v.at[b, :]], rows_v.at[b], gsem.at[b])

    def lap(g, carry):
        for b in range(NBUF):
            j = g * NBUF + b
            pltpu.make_async_copy(
                g_s.at[ridx_v.at[j, :]], rows_v.at[b], gsem.at[b]).wait()
            pltpu.async_copy(
                rows_v.at[b], acc.at[cidx_v.at[j, :]], ssem.at[b], add=True)
            bb = (b + LOOKAHEAD) % NBUF

            @pl.when(j + LOOKAHEAD < CPT)
            def _():
                @pl.when(j >= LOOKAHEAD)
                def _():
                    pltpu.make_async_copy(
                        rows_v.at[bb], acc.at[cidx_v.at[j, :]],
                        ssem.at[bb]).wait()

                pltpu.async_copy(
                    g_s.at[ridx_v.at[j + LOOKAHEAD, :]], rows_v.at[bb],
                    gsem.at[bb])
        return carry

    lax.fori_loop(0, CPT // NBUF, lap, 0)
    for b in range(NBUF):
        pltpu.make_async_copy(
            rows_v.at[b], acc.at[cidx_v.at[0, :]], ssem.at[b]).wait()

    @pl.when(wid < EXTRA)
    def _():
        pltpu.sync_copy(ridx_hbm.at[pl.ds(NC * NS * CPT + wid, 1)],
                        ridx_v.at[pl.ds(0, 1)])
        pltpu.sync_copy(cidx_hbm.at[pl.ds(NC * NS * CPT + wid, 1)],
                        cidx_v.at[pl.ds(0, 1)])
        pltpu.async_copy(g_s.at[ridx_v.at[0, :]], rows_v.at[0],
                         gsem.at[0]).wait()
        pltpu.sync_copy(rows_v.at[0], acc.at[cidx_v.at[0, :]], add=True)

    plsc.subcore_barrier()
    pltpu.sync_copy(acc.at[pl.ds(sid * ROWS_PER_TILE, ROWS_PER_TILE)], stage_v)
    pltpu.sync_copy(stage_v, out_hbm.at[cid, pl.ds(sid * ROWS_PER_TILE, ROWS_PER_TILE)])


_sc_params = pltpu.CompilerParams(use_tc_tiling_on_sc=False)

_deg_pass = functools.partial(
    pl.kernel, _deg_body,
    out_type=jax.ShapeDtypeStruct((NC, NODES_PAD), jnp.float32),
    mesh=_mesh,
    compiler_params=_sc_params,
    scratch_types=[
        pltpu.VMEM((CPT, CHUNK), jnp.int32),
        pltpu.VMEM((CHUNK,), jnp.float32),
        pltpu.VMEM((ROWS_PER_TILE,), jnp.float32),
        pltpu.VMEM_SHARED((NODES_PAD,), jnp.float32),
        pltpu.SemaphoreType.DMA,
    ],
)()

_hop_pass = functools.partial(
    pl.kernel, _hop_body,
    out_type=jax.ShapeDtypeStruct((NC, NODES_PAD, N_CLASSES), jnp.float32),
    mesh=_mesh,
    compiler_params=_sc_params,
    scratch_types=[
        pltpu.VMEM((CPT, CHUNK), jnp.int32),
        pltpu.VMEM((CPT, CHUNK), jnp.int32),
        pltpu.VMEM((NBUF, CHUNK, N_CLASSES), jnp.float32),
        pltpu.VMEM((ROWS_PER_TILE, N_CLASSES), jnp.float32),
        pltpu.VMEM((G_ROWS_PER_TILE, N_CLASSES), jnp.float32),
        pltpu.VMEM_SHARED((NODES_PAD, N_CLASSES), jnp.float32),
        pltpu.VMEM_SHARED((N_NODES, N_CLASSES), jnp.float32),
        (pltpu.SemaphoreType.DMA((NBUF,)), pltpu.SemaphoreType.DMA((NBUF,))),
    ],
)()


N_FLAT = N_NODES * N_CLASSES // 128
NP_FLAT = NODES_PAD * N_CLASSES // 128


def _matmul_body(x_ref, w_ref, y_ref):
    y_ref[...] = jnp.dot(x_ref[...], w_ref[...],
                         preferred_element_type=jnp.float32)


def _scale_in_body(y_ref, degp_ref, g1_ref, dinv_ref, invdeg_ref):
    deg = degp_ref[0, :N_FLAT] + degp_ref[1, :N_FLAT] + 1.0
    dinv = lax.rsqrt(deg)
    invdeg = 1.0 / deg
    g1_ref[...] = y_ref[...] * dinv
    dinv_ref[...] = dinv
    invdeg_ref[...] = invdeg


def _mid_body(sp_ref, g1_ref, invdeg_ref, g2_ref):
    s = sp_ref[0, :N_FLAT] + sp_ref[1, :N_FLAT] + g1_ref[...]
    g2_ref[...] = s * invdeg_ref[...]


def _out_body(sp_ref, g2_ref, dinv_ref, b_ref, out_ref):
    s = sp_ref[0, :N_FLAT] + sp_ref[1, :N_FLAT] + g2_ref[...]
    out_ref[...] = s * dinv_ref[...] + b_ref[...]


_flat_spec = pl.BlockSpec((N_FLAT, 128), lambda: (0, 0))
_partial_spec = pl.BlockSpec((NC, NP_FLAT, 128), lambda: (0, 0, 0))
_flat_out = jax.ShapeDtypeStruct((N_FLAT, 128), jnp.float32)

_matmul = pl.pallas_call(
    _matmul_body,
    in_specs=[pl.BlockSpec((N_FLAT, 8 * D_FEAT), lambda: (0, 0)),
              pl.BlockSpec((8 * D_FEAT, 128), lambda: (0, 0))],
    out_specs=_flat_spec,
    out_shape=_flat_out,
)

_scale_in = pl.pallas_call(
    _scale_in_body,
    in_specs=[_flat_spec, _partial_spec],
    out_specs=[_flat_spec] * 3,
    out_shape=[_flat_out] * 3,
)

_mid = pl.pallas_call(
    _mid_body,
    in_specs=[_partial_spec, _flat_spec, _flat_spec],
    out_specs=_flat_spec,
    out_shape=_flat_out,
)

_out_stage = pl.pallas_call(
    _out_body,
    in_specs=[_partial_spec, _flat_spec, _flat_spec,
              pl.BlockSpec((1, 128), lambda: (0, 0))],
    out_specs=_flat_spec,
    out_shape=_flat_out,
)


def kernel(x, edge_index, W, b):
    col_p = edge_index[1].astype(jnp.int32).reshape(NCHUNKS, CHUNK)
    row_src = lax.optimization_barrier(edge_index)
    row_p = row_src[0].astype(jnp.int32).reshape(NCHUNKS, CHUNK)

    ones_col = jnp.ones((CHUNK,), jnp.float32)
    zeros_col = jnp.zeros((ROWS_PER_TILE,), jnp.float32)
    zeros_rows = jnp.zeros((ROWS_PER_TILE, N_CLASSES), jnp.float32)
    b_flat = jnp.tile(b, 8).reshape(1, 128)

    degp = _deg_pass(col_p, ones_col, zeros_col)
    degp_flat = jnp.broadcast_to(
        degp.reshape(NC, NP_FLAT, 8, 1), (NC, NP_FLAT, 8, N_CLASSES)
    ).reshape(NC, NP_FLAT, 128)
    wb = jnp.kron(jnp.eye(8, dtype=jnp.float32), W)
    y = _matmul(x.reshape(N_FLAT, 8 * D_FEAT), wb)
    g1, dinv, invdeg = _scale_in(y, degp_flat)
    s1p = _hop_pass(row_p, col_p, g1.reshape(N_NODES, N_CLASSES), zeros_rows)
    g2 = _mid(s1p.reshape(NC, NP_FLAT, 128), g1, invdeg)
    s2p = _hop_pass(row_p, col_p, g2.reshape(N_NODES, N_CLASSES), zeros_rows)
    out = _out_stage(s2p.reshape(NC, NP_FLAT, 128), g2, dinv, b_flat)
    return out.reshape(N_NODES, N_CLASSES)


# speedup vs baseline: 1.1128x; 1.1128x over previous
import functools

import jax
import jax.numpy as jnp
from jax import lax
from jax.experimental import pallas as pl
from jax.experimental.pallas import tpu as pltpu
from jax.experimental.pallas import tpu_sc as plsc

N_NODES = 10000
N_EDGES = 320000
D_FEAT = 128
N_CLASSES = 16

NC = 2
NS = 16
CHUNK = 256
NCHUNKS = N_EDGES // CHUNK
CPT = 39
EXTRA = NCHUNKS - NC * NS * CPT
NODES_PAD = 10240
ROWS_PER_TILE = NODES_PAD // NS
G_ROWS_PER_TILE = N_NODES // NS

NBUF = 3
LOOKAHEAD = 2

_mesh = plsc.VectorSubcoreMesh(core_axis_name="c", subcore_axis_name="s")


def _deg_body(cidx_hbm, ones_hbm, zeros_hbm, out_hbm, cidx_v, ones_v, stage_v, acc, sem):
    cid = lax.axis_index("c")
    sid = lax.axis_index("s")
    wid = cid * NS + sid
    pltpu.sync_copy(zeros_hbm, stage_v)
    pltpu.sync_copy(stage_v, acc.at[pl.ds(sid * ROWS_PER_TILE, ROWS_PER_TILE)])
    pltpu.sync_copy(ones_hbm, ones_v)
    pltpu.sync_copy(cidx_hbm.at[pl.ds(wid * CPT, CPT)], cidx_v)
    plsc.subcore_barrier()

    def chunk(j, carry):
        pltpu.async_copy(ones_v, acc.at[cidx_v.at[j, :]], sem, add=True)
        return carry

    lax.fori_loop(0, CPT, chunk, 0)

    @pl.when(wid < EXTRA)
    def _():
        pltpu.sync_copy(cidx_hbm.at[pl.ds(NC * NS * CPT + wid, 1)],
                        cidx_v.at[pl.ds(0, 1)])
        pltpu.async_copy(ones_v, acc.at[cidx_v.at[0, :]], sem, add=True)

    def drain(j, carry):
        pltpu.make_async_copy(ones_v, acc.at[cidx_v.at[0, :]], sem).wait()
        return carry

    lax.fori_loop(0, CPT, drain, 0)

    @pl.when(wid < EXTRA)
    def _():
        pltpu.make_async_copy(ones_v, acc.at[cidx_v.at[0, :]], sem).wait()

    plsc.subcore_barrier()
    pltpu.sync_copy(acc.at[pl.ds(sid * ROWS_PER_TILE, ROWS_PER_TILE)], stage_v)
    pltpu.sync_copy(stage_v, out_hbm.at[cid, pl.ds(sid * ROWS_PER_TILE, ROWS_PER_TILE)])


def _hop_body(ridx_hbm, cidx_hbm, g_hbm, zeros_hbm, out_hbm,
              ridx_v, cidx_v, rows_v, stage_v, g_stage_v, acc, g_s, sem):
    cid = lax.axis_index("c")
    sid = lax.axis_index("s")
    wid = cid * NS + sid
    pltpu.sync_copy(zeros_hbm, stage_v)
    pltpu.sync_copy(stage_v, acc.at[pl.ds(sid * ROWS_PER_TILE, ROWS_PER_TILE)])
    pltpu.sync_copy(g_hbm.at[pl.ds(sid * G_ROWS_PER_TILE, G_ROWS_PER_TILE)],
                    g_stage_v)
    pltpu.sync_copy(g_stage_v,
                    g_s.at[pl.ds(sid * G_ROWS_PER_TILE, G_ROWS_PER_TILE)])
    pltpu.sync_copy(ridx_hbm.at[pl.ds(wid * CPT, CPT)], ridx_v)
    pltpu.sync_copy(cidx_hbm.at[pl.ds(wid * CPT, CPT)], cidx_v)
    gsem, ssem = sem
    plsc.subcore_barrier()

    for b in range(LOOKAHEAD):
        pltpu.async_copy(g_s.at[ridx_v.at[b, :]], rows_v.at[b], gsem.at[b])

    def lap(g, carry):
        for b in range(NBUF):
            j = g * NBUF + b
            pltpu.make_async_copy(
                g_s.at[ridx_v.at[j, :]], rows_v.at[b], gsem.at[b]).wait()
            pltpu.async_copy(
                rows_v.at[b], acc.at[cidx_v.at[j, :]], ssem.at[b], add=True)
            bb = (b + LOOKAHEAD) % NBUF

            @pl.when(j + LOOKAHEAD < CPT)
            def _():
                @pl.when(j >= LOOKAHEAD)
                def _():
                    pltpu.make_async_copy(
                        rows_v.at[bb], acc.at[cidx_v.at[j, :]],
                        ssem.at[bb]).wait()

                pltpu.async_copy(
                    g_s.at[ridx_v.at[j + LOOKAHEAD, :]], rows_v.at[bb],
                    gsem.at[bb])
        return carry

    lax.fori_loop(0, CPT // NBUF, lap, 0)
    for b in range(NBUF):
        pltpu.make_async_copy(
            rows_v.at[b], acc.at[cidx_v.at[0, :]], ssem.at[b]).wait()

    @pl.when(wid < EXTRA)
    def _():
        pltpu.sync_copy(ridx_hbm.at[pl.ds(NC * NS * CPT + wid, 1)],
                        ridx_v.at[pl.ds(0, 1)])
        pltpu.sync_copy(cidx_hbm.at[pl.ds(NC * NS * CPT + wid, 1)],
                        cidx_v.at[pl.ds(0, 1)])
        pltpu.async_copy(g_s.at[ridx_v.at[0, :]], rows_v.at[0],
                         gsem.at[0]).wait()
        pltpu.sync_copy(rows_v.at[0], acc.at[cidx_v.at[0, :]], add=True)

    plsc.subcore_barrier()
    pltpu.sync_copy(acc.at[pl.ds(sid * ROWS_PER_TILE, ROWS_PER_TILE)], stage_v)
    pltpu.sync_copy(stage_v, out_hbm.at[cid, pl.ds(sid * ROWS_PER_TILE, ROWS_PER_TILE)])


_sc_params = pltpu.CompilerParams(use_tc_tiling_on_sc=False)

_deg_pass = functools.partial(
    pl.kernel, _deg_body,
    out_type=jax.ShapeDtypeStruct((NC, NODES_PAD), jnp.float32),
    mesh=_mesh,
    compiler_params=_sc_params,
    scratch_types=[
        pltpu.VMEM((CPT, CHUNK), jnp.int32),
        pltpu.VMEM((CHUNK,), jnp.float32),
        pltpu.VMEM((ROWS_PER_TILE,), jnp.float32),
        pltpu.VMEM_SHARED((NODES_PAD,), jnp.float32),
        pltpu.SemaphoreType.DMA,
    ],
)()

_hop_pass = functools.partial(
    pl.kernel, _hop_body,
    out_type=jax.ShapeDtypeStruct((NC, NODES_PAD, N_CLASSES), jnp.float32),
    mesh=_mesh,
    compiler_params=_sc_params,
    scratch_types=[
        pltpu.VMEM((CPT, CHUNK), jnp.int32),
        pltpu.VMEM((CPT, CHUNK), jnp.int32),
        pltpu.VMEM((NBUF, CHUNK, N_CLASSES), jnp.float32),
        pltpu.VMEM((ROWS_PER_TILE, N_CLASSES), jnp.float32),
        pltpu.VMEM((G_ROWS_PER_TILE, N_CLASSES), jnp.float32),
        pltpu.VMEM_SHARED((NODES_PAD, N_CLASSES), jnp.float32),
        pltpu.VMEM_SHARED((N_NODES, N_CLASSES), jnp.float32),
        (pltpu.SemaphoreType.DMA((NBUF,)), pltpu.SemaphoreType.DMA((NBUF,))),
    ],
)()


N_FLAT = N_NODES * N_CLASSES // 128
NP_FLAT = NODES_PAD * N_CLASSES // 128


def _matmul_body(x_ref, w_ref, y_ref):
    y_ref[...] = jnp.dot(x_ref[...], w_ref[...],
                         preferred_element_type=jnp.float32)


def _scale_in_body(y_ref, degp_ref, g1_ref, dinv_ref, invdeg_ref):
    deg = degp_ref[0, :N_FLAT] + degp_ref[1, :N_FLAT] + 1.0
    dinv = lax.rsqrt(deg)
    invdeg = 1.0 / deg
    g1_ref[...] = y_ref[...] * dinv
    dinv_ref[...] = dinv
    invdeg_ref[...] = invdeg


def _mid_body(sp_ref, g1_ref, invdeg_ref, g2_ref):
    s = sp_ref[0, :N_FLAT] + sp_ref[1, :N_FLAT] + g1_ref[...]
    g2_ref[...] = s * invdeg_ref[...]


def _out_body(sp_ref, g2_ref, dinv_ref, b_ref, out_ref):
    s = sp_ref[0, :N_FLAT] + sp_ref[1, :N_FLAT] + g2_ref[...]
    out_ref[...] = s * dinv_ref[...] + b_ref[...]


_flat_spec = pl.BlockSpec((N_FLAT, 128), lambda: (0, 0))
_partial_spec = pl.BlockSpec((NC, NP_FLAT, 128), lambda: (0, 0, 0))
_flat_out = jax.ShapeDtypeStruct((N_FLAT, 128), jnp.float32)

_matmul = pl.pallas_call(
    _matmul_body,
    in_specs=[pl.BlockSpec((N_FLAT, 8 * D_FEAT), lambda: (0, 0)),
              pl.BlockSpec((8 * D_FEAT, 128), lambda: (0, 0))],
    out_specs=_flat_spec,
    out_shape=_flat_out,
)

_scale_in = pl.pallas_call(
    _scale_in_body,
    in_specs=[_flat_spec, _partial_spec],
    out_specs=[_flat_spec] * 3,
    out_shape=[_flat_out] * 3,
)

_mid = pl.pallas_call(
    _mid_body,
    in_specs=[_partial_spec, _flat_spec, _flat_spec],
    out_specs=_flat_spec,
    out_shape=_flat_out,
)

_out_stage = pl.pallas_call(
    _out_body,
    in_specs=[_partial_spec, _flat_spec, _flat_spec,
              pl.BlockSpec((1, 128), lambda: (0, 0))],
    out_specs=_flat_spec,
    out_shape=_flat_out,
)


def kernel(x, edge_index, W, b):
    row_p = edge_index[0].astype(jnp.int32).reshape(NCHUNKS, CHUNK)
    col_p = edge_index[1].astype(jnp.int32).reshape(NCHUNKS, CHUNK)

    ones_col = jnp.ones((CHUNK,), jnp.float32)
    zeros_col = jnp.zeros((ROWS_PER_TILE,), jnp.float32)
    zeros_rows = jnp.zeros((ROWS_PER_TILE, N_CLASSES), jnp.float32)
    b_flat = jnp.tile(b, 8).reshape(1, 128)

    degp = _deg_pass(col_p, ones_col, zeros_col)
    degp_flat = jnp.broadcast_to(
        degp.reshape(NC, NP_FLAT, 8, 1), (NC, NP_FLAT, 8, N_CLASSES)
    ).reshape(NC, NP_FLAT, 128)
    wb = jnp.kron(jnp.eye(8, dtype=jnp.float32), W)
    y = _matmul(x.reshape(N_FLAT, 8 * D_FEAT), wb)
    g1, dinv, invdeg = _scale_in(y, degp_flat)
    s1p = _hop_pass(row_p, col_p, g1.reshape(N_NODES, N_CLASSES), zeros_rows)
    g2 = _mid(s1p.reshape(NC, NP_FLAT, 128), g1, invdeg)
    s2p = _hop_pass(row_p, col_p, g2.reshape(N_NODES, N_CLASSES), zeros_rows)
    out = _out_stage(s2p.reshape(NC, NP_FLAT, 128), g2, dinv, b_flat)
    return out.reshape(N_NODES, N_CLASSES)


# speedup vs baseline: 1.3316x; 1.1966x over previous
import functools

import jax
import jax.numpy as jnp
from jax import lax
from jax.experimental import pallas as pl
from jax.experimental.pallas import tpu as pltpu
from jax.experimental.pallas import tpu_sc as plsc

N_NODES = 10000
N_EDGES = 320000
D_FEAT = 128
N_CLASSES = 16

NC = 2
NS = 16
CHUNK = 128
NCHUNKS = N_EDGES // CHUNK
CPT = 78
EXTRA = NCHUNKS - NC * NS * CPT
NODES_PAD = 10240
ROWS_PER_TILE = NODES_PAD // NS
G_ROWS_PER_TILE = N_NODES // NS

NBUF = 6
LOOKAHEAD = 3

_mesh = plsc.VectorSubcoreMesh(core_axis_name="c", subcore_axis_name="s")


def _deg_body(e_hbm, ones_hbm, zeros_hbm, out_hbm, eidx_v, ones_v, stage_v, acc, sem):
    cid = lax.axis_index("c")
    sid = lax.axis_index("s")
    wid = cid * NS + sid
    pltpu.sync_copy(zeros_hbm, stage_v)
    pltpu.sync_copy(stage_v, acc.at[pl.ds(sid * ROWS_PER_TILE, ROWS_PER_TILE)])
    pltpu.sync_copy(ones_hbm, ones_v)
    pltpu.sync_copy(e_hbm.at[pl.ds(wid * CPT, CPT)], eidx_v)
    plsc.subcore_barrier()

    def chunk(j, carry):
        pltpu.async_copy(ones_v, acc.at[eidx_v.at[j, 1, :]], sem, add=True)
        return carry

    lax.fori_loop(0, CPT, chunk, 0)

    @pl.when(wid < EXTRA)
    def _():
        pltpu.sync_copy(e_hbm.at[pl.ds(NC * NS * CPT + wid, 1)],
                        eidx_v.at[pl.ds(0, 1)])
        pltpu.async_copy(ones_v, acc.at[eidx_v.at[0, 1, :]], sem, add=True)

    def drain(j, carry):
        pltpu.make_async_copy(ones_v, acc.at[eidx_v.at[0, 1, :]], sem).wait()
        return carry

    lax.fori_loop(0, CPT, drain, 0)

    @pl.when(wid < EXTRA)
    def _():
        pltpu.make_async_copy(ones_v, acc.at[eidx_v.at[0, 1, :]], sem).wait()

    plsc.subcore_barrier()
    pltpu.sync_copy(acc.at[pl.ds(sid * ROWS_PER_TILE, ROWS_PER_TILE)], stage_v)
    pltpu.sync_copy(stage_v, out_hbm.at[cid, pl.ds(sid * ROWS_PER_TILE, ROWS_PER_TILE)])


def _hop_body(e_hbm, g_hbm, zeros_hbm, out_hbm,
              eidx_v, rows_v, stage_v, g_stage_v, acc, g_s, sem):
    cid = lax.axis_index("c")
    sid = lax.axis_index("s")
    wid = cid * NS + sid
    pltpu.sync_copy(zeros_hbm, stage_v)
    pltpu.sync_copy(stage_v, acc.at[pl.ds(sid * ROWS_PER_TILE, ROWS_PER_TILE)])
    pltpu.sync_copy(g_hbm.at[pl.ds(sid * G_ROWS_PER_TILE, G_ROWS_PER_TILE)],
                    g_stage_v)
    pltpu.sync_copy(g_stage_v,
                    g_s.at[pl.ds(sid * G_ROWS_PER_TILE, G_ROWS_PER_TILE)])
    pltpu.sync_copy(e_hbm.at[pl.ds(wid * CPT, CPT)], eidx_v)
    gsem, ssem = sem
    plsc.subcore_barrier()

    for b in range(LOOKAHEAD):
        pltpu.async_copy(g_s.at[eidx_v.at[b, 0, :]], rows_v.at[b], gsem.at[b])

    def lap(g, carry):
        for b in range(NBUF):
            j = g * NBUF + b
            pltpu.make_async_copy(
                g_s.at[eidx_v.at[j, 0, :]], rows_v.at[b], gsem.at[b]).wait()
            pltpu.async_copy(
                rows_v.at[b], acc.at[eidx_v.at[j, 1, :]], ssem.at[b],
                add=True)
            bb = (b + LOOKAHEAD) % NBUF

            @pl.when(j + LOOKAHEAD < CPT)
            def _():
                @pl.when(j >= LOOKAHEAD)
                def _():
                    pltpu.make_async_copy(
                        rows_v.at[bb], acc.at[eidx_v.at[j, 1, :]],
                        ssem.at[bb]).wait()

                pltpu.async_copy(
                    g_s.at[eidx_v.at[j + LOOKAHEAD, 0, :]], rows_v.at[bb],
                    gsem.at[bb])
        return carry

    lax.fori_loop(0, CPT // NBUF, lap, 0)
    for b in range(NBUF):
        pltpu.make_async_copy(
            rows_v.at[b], acc.at[eidx_v.at[0, 1, :]], ssem.at[b]).wait()

    @pl.when(wid < EXTRA)
    def _():
        pltpu.sync_copy(e_hbm.at[pl.ds(NC * NS * CPT + wid, 1)],
                        eidx_v.at[pl.ds(0, 1)])
        pltpu.async_copy(g_s.at[eidx_v.at[0, 0, :]], rows_v.at[0],
                         gsem.at[0]).wait()
        pltpu.sync_copy(rows_v.at[0], acc.at[eidx_v.at[0, 1, :]], add=True)

    plsc.subcore_barrier()
    pltpu.sync_copy(acc.at[pl.ds(sid * ROWS_PER_TILE, ROWS_PER_TILE)], stage_v)
    pltpu.sync_copy(stage_v, out_hbm.at[cid, pl.ds(sid * ROWS_PER_TILE, ROWS_PER_TILE)])


_sc_params = pltpu.CompilerParams(use_tc_tiling_on_sc=False)

_deg_pass = functools.partial(
    pl.kernel, _deg_body,
    out_type=jax.ShapeDtypeStruct((NC, NODES_PAD), jnp.float32),
    mesh=_mesh,
    compiler_params=_sc_params,
    scratch_types=[
        pltpu.VMEM((CPT, NC, CHUNK), jnp.int32),
        pltpu.VMEM((CHUNK,), jnp.float32),
        pltpu.VMEM((ROWS_PER_TILE,), jnp.float32),
        pltpu.VMEM_SHARED((NODES_PAD,), jnp.float32),
        pltpu.SemaphoreType.DMA,
    ],
)()

_hop_pass = functools.partial(
    pl.kernel, _hop_body,
    out_type=jax.ShapeDtypeStruct((NC, NODES_PAD, N_CLASSES), jnp.float32),
    mesh=_mesh,
    compiler_params=_sc_params,
    scratch_types=[
        pltpu.VMEM((CPT, NC, CHUNK), jnp.int32),
        pltpu.VMEM((NBUF, CHUNK, N_CLASSES), jnp.float32),
        pltpu.VMEM((ROWS_PER_TILE, N_CLASSES), jnp.float32),
        pltpu.VMEM((G_ROWS_PER_TILE, N_CLASSES), jnp.float32),
        pltpu.VMEM_SHARED((NODES_PAD, N_CLASSES), jnp.float32),
        pltpu.VMEM_SHARED((N_NODES, N_CLASSES), jnp.float32),
        (pltpu.SemaphoreType.DMA((NBUF,)), pltpu.SemaphoreType.DMA((NBUF,))),
    ],
)()


N_FLAT = N_NODES * N_CLASSES // 128
NP_FLAT = NODES_PAD * N_CLASSES // 128


def _matmul_body(x_ref, w_ref, y_ref):
    y_ref[...] = jnp.dot(x_ref[...], w_ref[...],
                         preferred_element_type=jnp.float32)


def _scale_in_body(y_ref, degp_ref, g1_ref, dinv_ref, invdeg_ref):
    deg = degp_ref[0, :N_FLAT] + degp_ref[1, :N_FLAT] + 1.0
    dinv = lax.rsqrt(deg)
    invdeg = 1.0 / deg
    g1_ref[...] = y_ref[...] * dinv
    dinv_ref[...] = dinv
    invdeg_ref[...] = invdeg


def _mid_body(sp_ref, g1_ref, invdeg_ref, g2_ref):
    s = sp_ref[0, :N_FLAT] + sp_ref[1, :N_FLAT] + g1_ref[...]
    g2_ref[...] = s * invdeg_ref[...]


def _out_body(sp_ref, g2_ref, dinv_ref, b_ref, out_ref):
    s = sp_ref[0, :N_FLAT] + sp_ref[1, :N_FLAT] + g2_ref[...]
    out_ref[...] = s * dinv_ref[...] + b_ref[...]


_flat_spec = pl.BlockSpec((N_FLAT, 128), lambda: (0, 0))
_partial_spec = pl.BlockSpec((NC, NP_FLAT, 128), lambda: (0, 0, 0))
_flat_out = jax.ShapeDtypeStruct((N_FLAT, 128), jnp.float32)

_matmul = pl.pallas_call(
    _matmul_body,
    in_specs=[pl.BlockSpec((N_FLAT, 8 * D_FEAT), lambda: (0, 0)),
              pl.BlockSpec((8 * D_FEAT, 128), lambda: (0, 0))],
    out_specs=_flat_spec,
    out_shape=_flat_out,
)

_scale_in = pl.pallas_call(
    _scale_in_body,
    in_specs=[_flat_spec, _partial_spec],
    out_specs=[_flat_spec] * 3,
    out_shape=[_flat_out] * 3,
)

_mid = pl.pallas_call(
    _mid_body,
    in_specs=[_partial_spec, _flat_spec, _flat_spec],
    out_specs=_flat_spec,
    out_shape=_flat_out,
)

_out_stage = pl.pallas_call(
    _out_body,
    in_specs=[_partial_spec, _flat_spec, _flat_spec,
              pl.BlockSpec((1, 128), lambda: (0, 0))],
    out_specs=_flat_spec,
    out_shape=_flat_out,
)


def kernel(x, edge_index, W, b):
    e_pairs = edge_index.astype(jnp.int32).reshape(
        2, NCHUNKS, CHUNK).transpose(1, 0, 2)

    ones_col = jnp.ones((CHUNK,), jnp.float32)
    zeros_col = jnp.zeros((ROWS_PER_TILE,), jnp.float32)
    zeros_rows = jnp.zeros((ROWS_PER_TILE, N_CLASSES), jnp.float32)
    b_flat = jnp.tile(b, 8).reshape(1, 128)

    degp = _deg_pass(e_pairs, ones_col, zeros_col)
    degp_flat = jnp.broadcast_to(
        degp.reshape(NC, NP_FLAT, 8, 1), (NC, NP_FLAT, 8, N_CLASSES)
    ).reshape(NC, NP_FLAT, 128)
    wb = jnp.kron(jnp.eye(8, dtype=jnp.float32), W)
    y = _matmul(x.reshape(N_FLAT, 8 * D_FEAT), wb)
    g1, dinv, invdeg = _scale_in(y, degp_flat)
    s1p = _hop_pass(e_pairs, g1.reshape(N_NODES, N_CLASSES), zeros_rows)
    g2 = _mid(s1p.reshape(NC, NP_FLAT, 128), g1, invdeg)
    s2p = _hop_pass(e_pairs, g2.reshape(N_NODES, N_CLASSES), zeros_rows)
    out = _out_stage(s2p.reshape(NC, NP_FLAT, 128), g2, dinv, b_flat)
    return out.reshape(N_NODES, N_CLASSES)
